# baseline (speedup 1.0000x reference)
import functools

import jax
import jax.numpy as jnp
from jax.experimental import pallas as pl
from jax.experimental.pallas import tpu as pltpu
from jax.experimental.pallas import tpu_sc as plsc

T, H = 2048, 768
E, K, F = 8, 2, 3072
BLK = 512
NB = (T * K) // BLK + E
CAP = NB * BLK
NC, NS = 2, 16
NW = NC * NS
TPW = T // NW
CHT = 32

_RW_PAD = 128


def _route_body(x_ref, wrt_ref, pos0_ref, pos1_ref, p0_ref, p1_ref,
                be_ref, act_ref):
    x = x_ref[...]
    logits = jnp.dot(x, wrt_ref[...], preferred_element_type=jnp.float32)
    l8 = logits[:, :E]
    iota_e = jax.lax.broadcasted_iota(jnp.int32, (T, E), 1)
    m1 = jnp.max(l8, axis=1, keepdims=True)
    i1 = jnp.min(jnp.where(l8 == m1, iota_e, E), axis=1, keepdims=True)
    l2 = jnp.where(iota_e == i1, -jnp.inf, l8)
    m2 = jnp.max(l2, axis=1, keepdims=True)
    i2 = jnp.min(jnp.where(l2 == m2, iota_e, E), axis=1, keepdims=True)
    ex = jnp.exp(m2 - m1)
    den = 1.0 + ex
    p0_ref[...] = 1.0 / den
    p1_ref[...] = ex / den

    one1 = (iota_e == i1).astype(jnp.float32)
    one2 = (iota_e == i2).astype(jnp.float32)
    cnt = one1 + one2

    CH = 128
    nch = T // CH
    r = jax.lax.broadcasted_iota(jnp.int32, (CH, CH), 0)
    c = jax.lax.broadcasted_iota(jnp.int32, (CH, CH), 1)
    ltri = (r > c).astype(jnp.float32)
    pin = []
    sums = []
    for ci in range(nch):
        blk = cnt[ci * CH:(ci + 1) * CH, :]
        pin.append(jnp.dot(ltri, blk, preferred_element_type=jnp.float32))
        sums.append(jnp.sum(blk, axis=0, keepdims=True))
    s = jnp.concatenate(sums, axis=0)
    r2 = jax.lax.broadcasted_iota(jnp.int32, (nch, nch), 0)
    c2 = jax.lax.broadcasted_iota(jnp.int32, (nch, nch), 1)
    ltri2 = (r2 > c2).astype(jnp.float32)
    off = jnp.dot(ltri2, s, preferred_element_type=jnp.float32)
    pre = jnp.concatenate(
        [pin[ci] + off[ci:ci + 1, :] for ci in range(nch)], axis=0)

    counts = jnp.sum(s, axis=0, keepdims=True)
    nb_blk = jnp.floor((counts + (BLK - 1)) * (1.0 / BLK)) * BLK
    re_ = jax.lax.broadcasted_iota(jnp.int32, (E, E), 0)
    ce_ = jax.lax.broadcasted_iota(jnp.int32, (E, E), 1)
    utri = (re_ < ce_).astype(jnp.float32)
    pstart = jnp.dot(nb_blk, utri, preferred_element_type=jnp.float32)
    total = jnp.sum(nb_blk)

    rank0 = jnp.sum(one1 * pre, axis=1, keepdims=True)
    rank1 = jnp.sum(one2 * pre, axis=1, keepdims=True)
    base0 = jnp.sum(one1 * pstart, axis=1, keepdims=True)
    base1 = jnp.sum(one2 * pstart, axis=1, keepdims=True)
    pos0_ref[...] = (base0 + rank0).astype(jnp.int32)
    pos1_ref[...] = (base1 + rank1).astype(jnp.int32)

    iob = jax.lax.broadcasted_iota(jnp.int32, (NB, 1), 0).astype(
        jnp.float32) * BLK
    bc = jnp.minimum(iob, total - BLK)
    ind = ((bc >= pstart) & (bc < pstart + nb_blk)).astype(jnp.float32)
    eids = jax.lax.broadcasted_iota(jnp.int32, (NB, E), 1).astype(jnp.float32)
    be_ref[...] = jnp.sum(ind * eids, axis=1, keepdims=True).astype(jnp.int32)
    act_ref[...] = (iob < total).astype(jnp.int32)


def _route_call(x, wrt, interpret=False):
    f32, i32 = jnp.float32, jnp.int32
    return pl.pallas_call(
        _route_body,
        out_shape=[
            jax.ShapeDtypeStruct((T, 1), i32),
            jax.ShapeDtypeStruct((T, 1), i32),
            jax.ShapeDtypeStruct((T, 1), f32),
            jax.ShapeDtypeStruct((T, 1), f32),
            jax.ShapeDtypeStruct((NB, 1), i32),
            jax.ShapeDtypeStruct((NB, 1), i32),
        ],
        interpret=interpret,
    )(x, wrt)


def _expert_call(be, nbact, xs, W1, b1, W2, b2, ps):
    f32 = jnp.float32

    def outer(be_ref, nb_ref, xs_hbm, w1_hbm, b1_hbm, w2_hbm, b2_hbm,
              ps_hbm, ys_hbm):
        def inner(xs_ref, w1_ref, b1_ref, w2_ref, b2_ref, ps_ref, ys_ref):
            xb = xs_ref[...]
            h = jnp.dot(xb, w1_ref[0], preferred_element_type=f32)
            h = h + b1_ref[0]
            h = 0.5 * h * (1.0 + jax.lax.erf(h * 0.7071067811865476))
            y = jnp.dot(h, w2_ref[0], preferred_element_type=f32)
            ys_ref[...] = (y + b2_ref[0]) * ps_ref[:, 0:1]

        look3 = pl.Buffered(buffer_count=3, use_lookahead=True)
        look = pl.Buffered(buffer_count=2, use_lookahead=True)
        pltpu.emit_pipeline(
            inner,
            grid=(nb_ref[0],),
            in_specs=[
                pl.BlockSpec((BLK, H), lambda b: (b, 0)),
                pl.BlockSpec((1, H, F), lambda b: (be_ref[b], 0, 0),
                             pipeline_mode=look3),
                pl.BlockSpec((1, 1, F), lambda b: (be_ref[b], 0, 0)),
                pl.BlockSpec((1, F, H), lambda b: (be_ref[b], 0, 0),
                             pipeline_mode=look),
                pl.BlockSpec((1, 1, H), lambda b: (be_ref[b], 0, 0)),
                pl.BlockSpec((BLK, 128), lambda b: (b, 0)),
            ],
            out_specs=[pl.BlockSpec((BLK, H), lambda b: (b, 0))],
        )(xs_hbm, w1_hbm, b1_hbm, w2_hbm, b2_hbm, ps_hbm, ys_hbm)

    sspec = pl.BlockSpec(memory_space=pltpu.SMEM)
    aspec = pl.BlockSpec(memory_space=pl.ANY)
    return pl.pallas_call(
        outer,
        in_specs=[sspec, sspec, aspec, aspec, aspec, aspec, aspec, aspec],
        out_specs=aspec,
        out_shape=jax.ShapeDtypeStruct((CAP, H), f32),
    )(be, nbact, xs, W1, b1.reshape(E, 1, F), W2, b2.reshape(E, 1, H), ps)


def _dispatch_call(x, pos0, pos1, p0r, p1r):
    mesh = plsc.VectorSubcoreMesh(core_axis_name="c", subcore_axis_name="s",
                                  num_cores=NC, num_subcores=NS)

    @functools.partial(
        pl.kernel,
        out_type=[
            jax.ShapeDtypeStruct((CAP, H), jnp.float32),
            jax.ShapeDtypeStruct((CAP, 128), jnp.float32),
        ],
        mesh=mesh,
        scratch_types=[
            pltpu.VMEM((TPW,), jnp.int32),
            pltpu.VMEM((TPW,), jnp.int32),
            pltpu.VMEM((TPW, H), jnp.float32),
            pltpu.VMEM((TPW, 128), jnp.float32),
            pltpu.VMEM((TPW, 128), jnp.float32),
            pltpu.SemaphoreType.DMA,
        ],
        compiler_params=pltpu.CompilerParams(use_tc_tiling_on_sc=True),
    )
    def k(x_hbm, pos0_hbm, pos1_hbm, p0_hbm, p1_hbm, xs_hbm, ps_hbm,
          i0_v, i1_v, rows_v, pv0, pv1, sem):
        wid = jax.lax.axis_index("s") * NC + jax.lax.axis_index("c")
        base = wid * TPW
        pltpu.sync_copy(pos0_hbm.at[pl.ds(base, TPW)], i0_v)
        pltpu.sync_copy(pos1_hbm.at[pl.ds(base, TPW)], i1_v)
        pltpu.sync_copy(x_hbm.at[pl.ds(base, TPW)], rows_v)
        pltpu.sync_copy(p0_hbm.at[pl.ds(base, TPW)], pv0)
        pltpu.sync_copy(p1_hbm.at[pl.ds(base, TPW)], pv1)
        c0 = pltpu.async_copy(rows_v, xs_hbm.at[i0_v], sem)
        c1 = pltpu.async_copy(rows_v, xs_hbm.at[i1_v], sem)
        c2 = pltpu.async_copy(pv0, ps_hbm.at[i0_v], sem)
        c3 = pltpu.async_copy(pv1, ps_hbm.at[i1_v], sem)
        c0.wait(); c1.wait(); c2.wait(); c3.wait()

    return k(x, pos0, pos1, p0r, p1r)


def _combine_call(ys, pos0r, pos1r):
    mesh = plsc.VectorSubcoreMesh(core_axis_name="c", subcore_axis_name="s",
                                  num_cores=NC, num_subcores=NS)
    nch = TPW // CHT

    @functools.partial(
        pl.kernel,
        out_type=jax.ShapeDtypeStruct((T, H), jnp.float32),
        mesh=mesh,
        scratch_types=[
            pltpu.VMEM((nch, CHT), jnp.int32),
            pltpu.VMEM((nch, CHT), jnp.int32),
            [pltpu.VMEM((CHT, H), jnp.float32) for _ in range(nch)],
            [pltpu.VMEM((CHT, H), jnp.float32) for _ in range(nch)],
            [pltpu.SemaphoreType.DMA for _ in range(nch)],
        ],
        compiler_params=pltpu.CompilerParams(use_tc_tiling_on_sc=True),
    )
    def k(ys_hbm, pos0_hbm, pos1_hbm, out_hbm, i0_v, i1_v, g0s, g1s, sems):
        wid = jax.lax.axis_index("s") * NC + jax.lax.axis_index("c")
        base = wid * TPW
        pltpu.sync_copy(pos0_hbm.at[wid], i0_v)
        pltpu.sync_copy(pos1_hbm.at[wid], i1_v)
        copies = []
        for ch in range(nch):
            copies.append(pltpu.async_copy(ys_hbm.at[i0_v.at[ch]], g0s[ch],
                                           sems[ch]))
            copies.append(pltpu.async_copy(ys_hbm.at[i1_v.at[ch]], g1s[ch],
                                           sems[ch]))
        for ch in range(nch):
            copies[2 * ch].wait()
            copies[2 * ch + 1].wait()
            g0_v, g1_v = g0s[ch], g1s[ch]

            def token_body(i, _):
                for j in range(H // 16):
                    a = g0_v[i, pl.ds(j * 16, 16)]
                    bb = g1_v[i, pl.ds(j * 16, 16)]
                    g0_v[i, pl.ds(j * 16, 16)] = a + bb
                return 0

            jax.lax.fori_loop(0, CHT, token_body, 0)
            pltpu.sync_copy(g0_v, out_hbm.at[pl.ds(base + ch * CHT, CHT)])

    return k(ys, pos0r, pos1r)


def kernel(hidden_states, router_weight, W1, b1, W2, b2):
    S_, B_, H_ = hidden_states.shape
    x = hidden_states.reshape(S_ * B_, H_)
    wrt = jnp.pad(router_weight, ((0, _RW_PAD - E), (0, 0))).T

    pos0c, pos1c, p0c, p1c, bec, actc = _route_call(x, wrt)
    pos0 = pos0c.reshape(T)
    pos1 = pos1c.reshape(T)
    be = bec.reshape(NB)
    act = actc.reshape(NB)

    p0r = jnp.broadcast_to(p0c, (T, 128))
    p1r = jnp.broadcast_to(p1c, (T, 128))
    xs, ps = _dispatch_call(x, pos0, pos1, p0r, p1r)
    nbact = jnp.sum(actc).reshape(1)
    ys = _expert_call(be, nbact, xs, W1, b1, W2, b2, ps)
    out = _combine_call(ys, pos0.reshape(NW, TPW // CHT, CHT),
                        pos1.reshape(NW, TPW // CHT, CHT))
    return out.reshape(S_, B_, H_)


# speedup vs baseline: 1.0134x; 1.0134x over previous
import functools

import jax
import jax.numpy as jnp
from jax.experimental import pallas as pl
from jax.experimental.pallas import tpu as pltpu
from jax.experimental.pallas import tpu_sc as plsc

T, H = 2048, 768
E, K, F = 8, 2, 3072
BLK = 512
NB = (T * K) // BLK + E
CAP = NB * BLK
NC, NS = 2, 16
NW = NC * NS
TPW = T // NW
CHT = 32

_RW_PAD = 128


def _route_body(x_ref, wrt_ref, pos0_ref, pos1_ref, p0_ref, p1_ref,
                be_ref, act_ref):
    x = x_ref[...]
    logits = jnp.dot(x, wrt_ref[...], preferred_element_type=jnp.float32)
    l8 = logits[:, :E]
    iota_e = jax.lax.broadcasted_iota(jnp.int32, (T, E), 1)
    m1 = jnp.max(l8, axis=1, keepdims=True)
    i1 = jnp.min(jnp.where(l8 == m1, iota_e, E), axis=1, keepdims=True)
    l2 = jnp.where(iota_e == i1, -jnp.inf, l8)
    m2 = jnp.max(l2, axis=1, keepdims=True)
    i2 = jnp.min(jnp.where(l2 == m2, iota_e, E), axis=1, keepdims=True)
    ex = jnp.exp(m2 - m1)
    den = 1.0 + ex
    p0_ref[...] = 1.0 / den
    p1_ref[...] = ex / den

    one1 = (iota_e == i1).astype(jnp.float32)
    one2 = (iota_e == i2).astype(jnp.float32)
    cnt = one1 + one2

    CH = 128
    nch = T // CH
    r = jax.lax.broadcasted_iota(jnp.int32, (CH, CH), 0)
    c = jax.lax.broadcasted_iota(jnp.int32, (CH, CH), 1)
    ltri = (r > c).astype(jnp.float32)
    pin = []
    sums = []
    for ci in range(nch):
        blk = cnt[ci * CH:(ci + 1) * CH, :]
        pin.append(jnp.dot(ltri, blk, preferred_element_type=jnp.float32))
        sums.append(jnp.sum(blk, axis=0, keepdims=True))
    s = jnp.concatenate(sums, axis=0)
    r2 = jax.lax.broadcasted_iota(jnp.int32, (nch, nch), 0)
    c2 = jax.lax.broadcasted_iota(jnp.int32, (nch, nch), 1)
    ltri2 = (r2 > c2).astype(jnp.float32)
    off = jnp.dot(ltri2, s, preferred_element_type=jnp.float32)
    pre = jnp.concatenate(
        [pin[ci] + off[ci:ci + 1, :] for ci in range(nch)], axis=0)

    counts = jnp.sum(s, axis=0, keepdims=True)
    nb_blk = jnp.floor((counts + (BLK - 1)) * (1.0 / BLK)) * BLK
    re_ = jax.lax.broadcasted_iota(jnp.int32, (E, E), 0)
    ce_ = jax.lax.broadcasted_iota(jnp.int32, (E, E), 1)
    utri = (re_ < ce_).astype(jnp.float32)
    pstart = jnp.dot(nb_blk, utri, preferred_element_type=jnp.float32)
    total = jnp.sum(nb_blk)

    rank0 = jnp.sum(one1 * pre, axis=1, keepdims=True)
    rank1 = jnp.sum(one2 * pre, axis=1, keepdims=True)
    base0 = jnp.sum(one1 * pstart, axis=1, keepdims=True)
    base1 = jnp.sum(one2 * pstart, axis=1, keepdims=True)
    pos0_ref[...] = (base0 + rank0).astype(jnp.int32)
    pos1_ref[...] = (base1 + rank1).astype(jnp.int32)

    iob = jax.lax.broadcasted_iota(jnp.int32, (NB, 1), 0).astype(
        jnp.float32) * BLK
    bc = jnp.minimum(iob, total - BLK)
    ind = ((bc >= pstart) & (bc < pstart + nb_blk)).astype(jnp.float32)
    eids = jax.lax.broadcasted_iota(jnp.int32, (NB, E), 1).astype(jnp.float32)
    be_ref[...] = jnp.sum(ind * eids, axis=1, keepdims=True).astype(jnp.int32)
    act_ref[...] = (iob < total).astype(jnp.int32)


def _route_call(x, wrt, interpret=False):
    f32, i32 = jnp.float32, jnp.int32
    return pl.pallas_call(
        _route_body,
        out_shape=[
            jax.ShapeDtypeStruct((T, 1), i32),
            jax.ShapeDtypeStruct((T, 1), i32),
            jax.ShapeDtypeStruct((T, 1), f32),
            jax.ShapeDtypeStruct((T, 1), f32),
            jax.ShapeDtypeStruct((NB, 1), i32),
            jax.ShapeDtypeStruct((NB, 1), i32),
        ],
        interpret=interpret,
    )(x, wrt)


def _expert_call(be, nbact, xs, W1, b1, W2, b2, ps):
    f32 = jnp.float32

    def outer(be_ref, nb_ref, xs_hbm, w1_hbm, b1_hbm, w2_hbm, b2_hbm,
              ps_hbm, ys_hbm):
        def inner(xs_ref, w1_ref, b1_ref, w2_ref, b2_ref, ps_ref, ys_ref):
            xb = xs_ref[...]
            h = jnp.dot(xb, w1_ref[0], preferred_element_type=f32)
            h = h + b1_ref[0]
            h = 0.5 * h * (1.0 + jax.lax.erf(h * 0.7071067811865476))
            y = jnp.dot(h, w2_ref[0], preferred_element_type=f32)
            ys_ref[...] = (y + b2_ref[0]) * ps_ref[:, 0:1]

        look = pl.Buffered(buffer_count=2, use_lookahead=True)
        pltpu.emit_pipeline(
            inner,
            grid=(nb_ref[0],),
            in_specs=[
                pl.BlockSpec((BLK, H), lambda b: (b, 0)),
                pl.BlockSpec((1, H, F), lambda b: (be_ref[b], 0, 0),
                             pipeline_mode=look),
                pl.BlockSpec((1, 1, F), lambda b: (be_ref[b], 0, 0)),
                pl.BlockSpec((1, F, H), lambda b: (be_ref[b], 0, 0),
                             pipeline_mode=look),
                pl.BlockSpec((1, 1, H), lambda b: (be_ref[b], 0, 0)),
                pl.BlockSpec((BLK, 128), lambda b: (b, 0)),
            ],
            out_specs=[pl.BlockSpec((BLK, H), lambda b: (b, 0))],
        )(xs_hbm, w1_hbm, b1_hbm, w2_hbm, b2_hbm, ps_hbm, ys_hbm)

    sspec = pl.BlockSpec(memory_space=pltpu.SMEM)
    aspec = pl.BlockSpec(memory_space=pl.ANY)
    return pl.pallas_call(
        outer,
        in_specs=[sspec, sspec, aspec, aspec, aspec, aspec, aspec, aspec],
        out_specs=aspec,
        out_shape=jax.ShapeDtypeStruct((CAP, H), f32),
    )(be, nbact, xs, W1, b1.reshape(E, 1, F), W2, b2.reshape(E, 1, H), ps)


def _dispatch_call(x, pos0, pos1, p0r, p1r):
    mesh = plsc.VectorSubcoreMesh(core_axis_name="c", subcore_axis_name="s",
                                  num_cores=NC, num_subcores=NS)

    @functools.partial(
        pl.kernel,
        out_type=[
            jax.ShapeDtypeStruct((CAP, H), jnp.float32),
            jax.ShapeDtypeStruct((CAP, 128), jnp.float32),
        ],
        mesh=mesh,
        scratch_types=[
            pltpu.VMEM((TPW,), jnp.int32),
            pltpu.VMEM((TPW,), jnp.int32),
            pltpu.VMEM((TPW, H), jnp.float32),
            pltpu.VMEM((TPW, 128), jnp.float32),
            pltpu.VMEM((TPW, 128), jnp.float32),
            pltpu.SemaphoreType.DMA,
        ],
        compiler_params=pltpu.CompilerParams(use_tc_tiling_on_sc=True),
    )
    def k(x_hbm, pos0_hbm, pos1_hbm, p0_hbm, p1_hbm, xs_hbm, ps_hbm,
          i0_v, i1_v, rows_v, pv0, pv1, sem):
        wid = jax.lax.axis_index("s") * NC + jax.lax.axis_index("c")
        base = wid * TPW
        pltpu.sync_copy(pos0_hbm.at[pl.ds(base, TPW)], i0_v)
        pltpu.sync_copy(pos1_hbm.at[pl.ds(base, TPW)], i1_v)
        pltpu.sync_copy(x_hbm.at[pl.ds(base, TPW)], rows_v)
        pltpu.sync_copy(p0_hbm.at[pl.ds(base, TPW)], pv0)
        pltpu.sync_copy(p1_hbm.at[pl.ds(base, TPW)], pv1)
        c0 = pltpu.async_copy(rows_v, xs_hbm.at[i0_v], sem)
        c1 = pltpu.async_copy(rows_v, xs_hbm.at[i1_v], sem)
        c2 = pltpu.async_copy(pv0, ps_hbm.at[i0_v], sem)
        c3 = pltpu.async_copy(pv1, ps_hbm.at[i1_v], sem)
        c0.wait(); c1.wait(); c2.wait(); c3.wait()

    return k(x, pos0, pos1, p0r, p1r)


def _combine_call(ys, pos0r, pos1r):
    mesh = plsc.VectorSubcoreMesh(core_axis_name="c", subcore_axis_name="s",
                                  num_cores=NC, num_subcores=NS)
    nch = TPW // CHT

    @functools.partial(
        pl.kernel,
        out_type=jax.ShapeDtypeStruct((T, H), jnp.float32),
        mesh=mesh,
        scratch_types=[
            pltpu.VMEM((nch, CHT), jnp.int32),
            pltpu.VMEM((nch, CHT), jnp.int32),
            [pltpu.VMEM((CHT, H), jnp.float32) for _ in range(nch)],
            [pltpu.VMEM((CHT, H), jnp.float32) for _ in range(nch)],
            [pltpu.SemaphoreType.DMA for _ in range(nch)],
        ],
        compiler_params=pltpu.CompilerParams(use_tc_tiling_on_sc=True),
    )
    def k(ys_hbm, pos0_hbm, pos1_hbm, out_hbm, i0_v, i1_v, g0s, g1s, sems):
        wid = jax.lax.axis_index("s") * NC + jax.lax.axis_index("c")
        base = wid * TPW
        pltpu.sync_copy(pos0_hbm.at[wid], i0_v)
        pltpu.sync_copy(pos1_hbm.at[wid], i1_v)
        copies = []
        for ch in range(nch):
            copies.append(pltpu.async_copy(ys_hbm.at[i0_v.at[ch]], g0s[ch],
                                           sems[ch]))
            copies.append(pltpu.async_copy(ys_hbm.at[i1_v.at[ch]], g1s[ch],
                                           sems[ch]))
        for ch in range(nch):
            copies[2 * ch].wait()
            copies[2 * ch + 1].wait()
            g0_v, g1_v = g0s[ch], g1s[ch]

            def token_body(i, _):
                for j in range(H // 16):
                    a = g0_v[i, pl.ds(j * 16, 16)]
                    bb = g1_v[i, pl.ds(j * 16, 16)]
                    g0_v[i, pl.ds(j * 16, 16)] = a + bb
                return 0

            jax.lax.fori_loop(0, CHT, token_body, 0)
            pltpu.sync_copy(g0_v, out_hbm.at[pl.ds(base + ch * CHT, CHT)])

    return k(ys, pos0r, pos1r)


def kernel(hidden_states, router_weight, W1, b1, W2, b2):
    S_, B_, H_ = hidden_states.shape
    x = hidden_states.reshape(S_ * B_, H_)
    wrt = jnp.pad(router_weight, ((0, _RW_PAD - E), (0, 0))).T

    pos0c, pos1c, p0c, p1c, bec, actc = _route_call(x, wrt)
    pos0 = pos0c.reshape(T)
    pos1 = pos1c.reshape(T)
    be = bec.reshape(NB)
    act = actc.reshape(NB)

    p0r = jnp.broadcast_to(p0c, (T, 128))
    p1r = jnp.broadcast_to(p1c, (T, 128))
    xs, ps = _dispatch_call(x, pos0, pos1, p0r, p1r)
    nbact = jnp.sum(actc).reshape(1)
    ys = _expert_call(be, nbact, xs, W1, b1, W2, b2, ps)
    out = _combine_call(ys, pos0.reshape(NW, TPW // CHT, CHT),
                        pos1.reshape(NW, TPW // CHT, CHT))
    return out.reshape(S_, B_, H_)


# speedup vs baseline: 1.0329x; 1.0193x over previous
import functools

import jax
import jax.numpy as jnp
from jax.experimental import pallas as pl
from jax.experimental.pallas import tpu as pltpu
from jax.experimental.pallas import tpu_sc as plsc

T, H = 2048, 768
E, K, F = 8, 2, 3072
BLK = 512
NB = (T * K) // BLK + E
CAP = NB * BLK
NC, NS = 2, 16
NW = NC * NS
TPW = T // NW
CHT = 32

_RW_PAD = 128


def _route_body(x_ref, wrt_ref, pos0_ref, pos1_ref, p0_ref, p1_ref,
                be_ref, act_ref, xp_ref):
    x = x_ref[...]
    xr = x.astype(jnp.bfloat16).astype(jnp.float32)
    u = jax.lax.bitcast_convert_type(xr, jnp.uint32)
    packed = (u[:, :H // 2] & jnp.uint32(0xFFFF0000)) | (u[:, H // 2:] >> 16)
    xp_ref[...] = jax.lax.bitcast_convert_type(packed, jnp.float32)
    logits = jnp.dot(x, wrt_ref[...], preferred_element_type=jnp.float32)
    l8 = logits[:, :E]
    iota_e = jax.lax.broadcasted_iota(jnp.int32, (T, E), 1)
    m1 = jnp.max(l8, axis=1, keepdims=True)
    i1 = jnp.min(jnp.where(l8 == m1, iota_e, E), axis=1, keepdims=True)
    l2 = jnp.where(iota_e == i1, -jnp.inf, l8)
    m2 = jnp.max(l2, axis=1, keepdims=True)
    i2 = jnp.min(jnp.where(l2 == m2, iota_e, E), axis=1, keepdims=True)
    ex = jnp.exp(m2 - m1)
    den = 1.0 + ex
    p0_ref[...] = 1.0 / den
    p1_ref[...] = ex / den

    one1 = (iota_e == i1).astype(jnp.float32)
    one2 = (iota_e == i2).astype(jnp.float32)
    cnt = one1 + one2

    CH = 128
    nch = T // CH
    r = jax.lax.broadcasted_iota(jnp.int32, (CH, CH), 0)
    c = jax.lax.broadcasted_iota(jnp.int32, (CH, CH), 1)
    ltri = (r > c).astype(jnp.float32)
    pin = []
    sums = []
    for ci in range(nch):
        blk = cnt[ci * CH:(ci + 1) * CH, :]
        pin.append(jnp.dot(ltri, blk, preferred_element_type=jnp.float32))
        sums.append(jnp.sum(blk, axis=0, keepdims=True))
    s = jnp.concatenate(sums, axis=0)
    r2 = jax.lax.broadcasted_iota(jnp.int32, (nch, nch), 0)
    c2 = jax.lax.broadcasted_iota(jnp.int32, (nch, nch), 1)
    ltri2 = (r2 > c2).astype(jnp.float32)
    off = jnp.dot(ltri2, s, preferred_element_type=jnp.float32)
    pre = jnp.concatenate(
        [pin[ci] + off[ci:ci + 1, :] for ci in range(nch)], axis=0)

    counts = jnp.sum(s, axis=0, keepdims=True)
    nb_blk = jnp.floor((counts + (BLK - 1)) * (1.0 / BLK)) * BLK
    re_ = jax.lax.broadcasted_iota(jnp.int32, (E, E), 0)
    ce_ = jax.lax.broadcasted_iota(jnp.int32, (E, E), 1)
    utri = (re_ < ce_).astype(jnp.float32)
    pstart = jnp.dot(nb_blk, utri, preferred_element_type=jnp.float32)
    total = jnp.sum(nb_blk)

    rank0 = jnp.sum(one1 * pre, axis=1, keepdims=True)
    rank1 = jnp.sum(one2 * pre, axis=1, keepdims=True)
    base0 = jnp.sum(one1 * pstart, axis=1, keepdims=True)
    base1 = jnp.sum(one2 * pstart, axis=1, keepdims=True)
    pos0_ref[...] = (base0 + rank0).astype(jnp.int32)
    pos1_ref[...] = (base1 + rank1).astype(jnp.int32)

    iob = jax.lax.broadcasted_iota(jnp.int32, (NB, 1), 0).astype(
        jnp.float32) * BLK
    bc = jnp.minimum(iob, total - BLK)
    ind = ((bc >= pstart) & (bc < pstart + nb_blk)).astype(jnp.float32)
    eids = jax.lax.broadcasted_iota(jnp.int32, (NB, E), 1).astype(jnp.float32)
    be_ref[...] = jnp.sum(ind * eids, axis=1, keepdims=True).astype(jnp.int32)
    act_ref[...] = (iob < total).astype(jnp.int32)


def _route_call(x, wrt, interpret=False):
    f32, i32 = jnp.float32, jnp.int32
    return pl.pallas_call(
        _route_body,
        out_shape=[
            jax.ShapeDtypeStruct((T, 1), i32),
            jax.ShapeDtypeStruct((T, 1), i32),
            jax.ShapeDtypeStruct((T, 1), f32),
            jax.ShapeDtypeStruct((T, 1), f32),
            jax.ShapeDtypeStruct((NB, 1), i32),
            jax.ShapeDtypeStruct((NB, 1), i32),
            jax.ShapeDtypeStruct((T, H // 2), f32),
        ],
        interpret=interpret,
    )(x, wrt)


def _expert_call(be, nbact, xs, W1, b1, W2, b2, ps):
    f32 = jnp.float32

    def outer(be_ref, nb_ref, xs_hbm, w1_hbm, b1_hbm, w2_hbm, b2_hbm,
              ps_hbm, ys_hbm):
        def inner(xs_ref, w1_ref, b1_ref, w2_ref, b2_ref, ps_ref, ys_ref):
            u = jax.lax.bitcast_convert_type(xs_ref[...], jnp.uint32)
            xa = jax.lax.bitcast_convert_type(
                u & jnp.uint32(0xFFFF0000), f32)
            xc = jax.lax.bitcast_convert_type(u << 16, f32)
            xb = jnp.concatenate([xa, xc], axis=1)
            h = jnp.dot(xb, w1_ref[0], preferred_element_type=f32)
            h = h + b1_ref[0]
            h = 0.5 * h * (1.0 + jax.lax.erf(h * 0.7071067811865476))
            y = jnp.dot(h, w2_ref[0], preferred_element_type=f32)
            ys_ref[...] = (y + b2_ref[0]) * ps_ref[:, 0:1]

        look = pl.Buffered(buffer_count=2, use_lookahead=True)
        pltpu.emit_pipeline(
            inner,
            grid=(nb_ref[0],),
            in_specs=[
                pl.BlockSpec((BLK, H // 2), lambda b: (b, 0)),
                pl.BlockSpec((1, H, F), lambda b: (be_ref[b], 0, 0),
                             pipeline_mode=look),
                pl.BlockSpec((1, 1, F), lambda b: (be_ref[b], 0, 0)),
                pl.BlockSpec((1, F, H), lambda b: (be_ref[b], 0, 0),
                             pipeline_mode=look),
                pl.BlockSpec((1, 1, H), lambda b: (be_ref[b], 0, 0)),
                pl.BlockSpec((BLK, 128), lambda b: (b, 0)),
            ],
            out_specs=[pl.BlockSpec((BLK, H), lambda b: (b, 0))],
        )(xs_hbm, w1_hbm, b1_hbm, w2_hbm, b2_hbm, ps_hbm, ys_hbm)

    sspec = pl.BlockSpec(memory_space=pltpu.SMEM)
    aspec = pl.BlockSpec(memory_space=pl.ANY)
    return pl.pallas_call(
        outer,
        in_specs=[sspec, sspec, aspec, aspec, aspec, aspec, aspec, aspec],
        out_specs=aspec,
        out_shape=jax.ShapeDtypeStruct((CAP, H), f32),
    )(be, nbact, xs, W1, b1.reshape(E, 1, F), W2, b2.reshape(E, 1, H), ps)


def _dispatch_call(x, pos0, pos1, p0r, p1r):
    mesh = plsc.VectorSubcoreMesh(core_axis_name="c", subcore_axis_name="s",
                                  num_cores=NC, num_subcores=NS)

    @functools.partial(
        pl.kernel,
        out_type=[
            jax.ShapeDtypeStruct((CAP, H // 2), jnp.float32),
            jax.ShapeDtypeStruct((CAP, 128), jnp.float32),
        ],
        mesh=mesh,
        scratch_types=[
            pltpu.VMEM((TPW,), jnp.int32),
            pltpu.VMEM((TPW,), jnp.int32),
            pltpu.VMEM((TPW, H // 2), jnp.float32),
            pltpu.VMEM((TPW, 128), jnp.float32),
            pltpu.VMEM((TPW, 128), jnp.float32),
            pltpu.SemaphoreType.DMA,
        ],
        compiler_params=pltpu.CompilerParams(use_tc_tiling_on_sc=True),
    )
    def k(x_hbm, pos0_hbm, pos1_hbm, p0_hbm, p1_hbm, xs_hbm, ps_hbm,
          i0_v, i1_v, rows_v, pv0, pv1, sem):
        wid = jax.lax.axis_index("s") * NC + jax.lax.axis_index("c")
        base = wid * TPW
        pltpu.sync_copy(pos0_hbm.at[pl.ds(base, TPW)], i0_v)
        pltpu.sync_copy(pos1_hbm.at[pl.ds(base, TPW)], i1_v)
        pltpu.sync_copy(x_hbm.at[pl.ds(base, TPW)], rows_v)
        pltpu.sync_copy(p0_hbm.at[pl.ds(base, TPW)], pv0)
        pltpu.sync_copy(p1_hbm.at[pl.ds(base, TPW)], pv1)
        c0 = pltpu.async_copy(rows_v, xs_hbm.at[i0_v], sem)
        c1 = pltpu.async_copy(rows_v, xs_hbm.at[i1_v], sem)
        c2 = pltpu.async_copy(pv0, ps_hbm.at[i0_v], sem)
        c3 = pltpu.async_copy(pv1, ps_hbm.at[i1_v], sem)
        c0.wait(); c1.wait(); c2.wait(); c3.wait()

    return k(x, pos0, pos1, p0r, p1r)


def _combine_call(ys, pos0r, pos1r):
    mesh = plsc.VectorSubcoreMesh(core_axis_name="c", subcore_axis_name="s",
                                  num_cores=NC, num_subcores=NS)
    nch = TPW // CHT

    @functools.partial(
        pl.kernel,
        out_type=jax.ShapeDtypeStruct((T, H), jnp.float32),
        mesh=mesh,
        scratch_types=[
            pltpu.VMEM((nch, CHT), jnp.int32),
            pltpu.VMEM((nch, CHT), jnp.int32),
            [pltpu.VMEM((CHT, H), jnp.float32) for _ in range(nch)],
            [pltpu.VMEM((CHT, H), jnp.float32) for _ in range(nch)],
            [pltpu.SemaphoreType.DMA for _ in range(nch)],
        ],
        compiler_params=pltpu.CompilerParams(use_tc_tiling_on_sc=True),
    )
    def k(ys_hbm, pos0_hbm, pos1_hbm, out_hbm, i0_v, i1_v, g0s, g1s, sems):
        wid = jax.lax.axis_index("s") * NC + jax.lax.axis_index("c")
        base = wid * TPW
        pltpu.sync_copy(pos0_hbm.at[wid], i0_v)
        pltpu.sync_copy(pos1_hbm.at[wid], i1_v)
        copies = []
        for ch in range(nch):
            copies.append(pltpu.async_copy(ys_hbm.at[i0_v.at[ch]], g0s[ch],
                                           sems[ch]))
            copies.append(pltpu.async_copy(ys_hbm.at[i1_v.at[ch]], g1s[ch],
                                           sems[ch]))
        for ch in range(nch):
            copies[2 * ch].wait()
            copies[2 * ch + 1].wait()
            g0_v, g1_v = g0s[ch], g1s[ch]

            def token_body(i, _):
                for j in range(H // 16):
                    a = g0_v[i, pl.ds(j * 16, 16)]
                    bb = g1_v[i, pl.ds(j * 16, 16)]
                    g0_v[i, pl.ds(j * 16, 16)] = a + bb
                return 0

            jax.lax.fori_loop(0, CHT, token_body, 0)
            pltpu.sync_copy(g0_v, out_hbm.at[pl.ds(base + ch * CHT, CHT)])

    return k(ys, pos0r, pos1r)


def kernel(hidden_states, router_weight, W1, b1, W2, b2):
    S_, B_, H_ = hidden_states.shape
    x = hidden_states.reshape(S_ * B_, H_)
    wrt = jnp.pad(router_weight, ((0, _RW_PAD - E), (0, 0))).T

    pos0c, pos1c, p0c, p1c, bec, actc, xp = _route_call(x, wrt)
    pos0 = pos0c.reshape(T)
    pos1 = pos1c.reshape(T)
    be = bec.reshape(NB)
    act = actc.reshape(NB)

    p0r = jnp.broadcast_to(p0c, (T, 128))
    p1r = jnp.broadcast_to(p1c, (T, 128))
    xs, ps = _dispatch_call(xp, pos0, pos1, p0r, p1r)
    nbact = jnp.sum(actc).reshape(1)
    ys = _expert_call(be, nbact, xs, W1, b1, W2, b2, ps)
    out = _combine_call(ys, pos0.reshape(NW, TPW // CHT, CHT),
                        pos1.reshape(NW, TPW // CHT, CHT))
    return out.reshape(S_, B_, H_)
